# Initial kernel scaffold; baseline (speedup 1.0000x reference)
import jax
import jax.numpy as jnp
from jax.experimental import pallas as pl


def kernel(x, codebook):
    raise NotImplementedError("write your pallas kernel here")


# speedup vs baseline: 1.8530x; 1.8530x over previous
import jax
import jax.numpy as jnp
from jax import lax
from jax.experimental import pallas as pl
from jax.experimental.pallas import tpu as pltpu
from jax.experimental.pallas import tpu_sc as plsc

_B, _T, _D = 16, 576, 64
_K = 1024
_N = _B * _T
_TILE = 1152
_NTILES = _N // _TILE
_COMMITMENT_COST = 0.25
_ENTROPY_LOSS_RATIO = 0.1

_NW = 32
_ROWS_PER_W = _N // _NW
_CHUNK = 96
_NCHUNK = _ROWS_PER_W // _CHUNK


def _vq_tc_body(x_ref, cb_ref, idx_ref, loss_ref, accp_ref, accs_ref):
    i = pl.program_id(0)
    x = x_ref[...]
    cb = cb_ref[...]
    mm = lax.dot_general(x, cb, (((1,), (1,)), ((), ())),
                         preferred_element_type=jnp.float32)
    x2 = jnp.sum(x * x, axis=1, keepdims=True)
    c2 = jnp.sum(cb * cb, axis=1)[None, :]
    d2 = jnp.maximum(x2 + c2 - 2.0 * mm, 0.0)
    d = jnp.sqrt(d2)

    dmin = jnp.min(d, axis=1, keepdims=True)
    kio = lax.broadcasted_iota(jnp.int32, (_TILE, _K), 1)
    idx = jnp.min(jnp.where(d == dmin, kio, _K), axis=1)
    idx_ref[...] = idx.reshape(1, 1, _TILE)

    sh = dmin - d
    p = jnp.exp(sh)
    s = jnp.sum(p, axis=1, keepdims=True)
    probs = p / s
    log_probs = sh - jnp.log(s)
    samp = jnp.sum(probs * log_probs)
    sq = jnp.sum(jnp.min(d2, axis=1))
    pcol = jnp.sum(probs, axis=0, keepdims=True)

    @pl.when(i == 0)
    def _():
        accp_ref[...] = jnp.zeros_like(accp_ref)
        accs_ref[0] = 0.0
        accs_ref[1] = 0.0

    accp_ref[...] += pcol
    accs_ref[0] += sq
    accs_ref[1] += samp

    @pl.when(i == _NTILES - 1)
    def _():
        avgp = accp_ref[...] / _N
        avg_ent = -jnp.sum(avgp * jnp.log(avgp + 1e-5))
        sample_ent = -(accs_ref[1] / _N)
        latent = (1.0 + _COMMITMENT_COST) * (accs_ref[0] / (_N * _D))
        loss = latent + _ENTROPY_LOSS_RATIO * (sample_ent - avg_ent)
        loss_ref[...] = jnp.full((1, 128), loss, jnp.float32)


_vq_tc = pl.pallas_call(
    _vq_tc_body,
    grid=(_NTILES,),
    in_specs=[
        pl.BlockSpec((_TILE, _D), lambda i: (i, 0)),
        pl.BlockSpec((_K, _D), lambda i: (0, 0)),
    ],
    out_specs=[
        pl.BlockSpec((1, 1, _TILE), lambda i: (i, 0, 0)),
        pl.BlockSpec((1, 128), lambda i: (0, 0)),
    ],
    out_shape=[
        jax.ShapeDtypeStruct((_NTILES, 1, _TILE), jnp.int32),
        jax.ShapeDtypeStruct((1, 128), jnp.float32),
    ],
    scratch_shapes=[
        pltpu.VMEM((1, _K), jnp.float32),
        pltpu.SMEM((2,), jnp.float32),
    ],
)


_DPAD = 128


def _sc_gather_body(cb_hbm, idx_hbm, out_hbm, idx_v, rows_v, sem):
    c = lax.axis_index("c")
    s = lax.axis_index("s")
    wid = s * 2 + c
    base = wid * _ROWS_PER_W
    pltpu.sync_copy(idx_hbm.at[wid], idx_v)
    for j in range(_NCHUNK):
        pltpu.async_copy(cb_hbm.at[idx_v.at[j]], rows_v, sem).wait()
        pltpu.sync_copy(rows_v, out_hbm.at[pl.ds(base + j * _CHUNK, _CHUNK)])


_sc_gather_cache = []


def _sc_gather(codebook, idx3):
    if not _sc_gather_cache:
        _sc_gather_cache.append(pl.kernel(
            _sc_gather_body,
            out_type=jax.ShapeDtypeStruct((_N, _DPAD), jnp.float32),
            mesh=plsc.VectorSubcoreMesh(core_axis_name="c",
                                        subcore_axis_name="s"),
            scratch_types=[
                pltpu.VMEM((_NCHUNK, _CHUNK), jnp.int32),
                pltpu.VMEM((_CHUNK, _DPAD), jnp.float32),
                pltpu.SemaphoreType.DMA,
            ],
        ))
    return _sc_gather_cache[0](codebook, idx3)


def kernel(x, codebook):
    flat_x = x.reshape(_N, _D)
    idx3, loss_out = _vq_tc(flat_x, codebook)
    idx = idx3.reshape(_N)
    cb_pad = jnp.concatenate(
        [codebook, jnp.zeros((_K, _DPAD - _D), jnp.float32)], axis=1)
    quantized = _sc_gather(cb_pad, idx.reshape(_NW, _NCHUNK, _CHUNK))
    return quantized[:, :_D].reshape(x.shape), loss_out[0, 0], idx


# speedup vs baseline: 1.9195x; 1.0358x over previous
import jax
import jax.numpy as jnp
from jax import lax
from jax.experimental import pallas as pl
from jax.experimental.pallas import tpu as pltpu
from jax.experimental.pallas import tpu_sc as plsc

_B, _T, _D = 16, 576, 64
_K = 1024
_N = _B * _T
_TILE = 1152
_NTILES = _N // _TILE
_COMMITMENT_COST = 0.25
_ENTROPY_LOSS_RATIO = 0.1

_NW = 32
_ROWS_PER_W = _N // _NW
_CHUNK = 96
_NCHUNK = _ROWS_PER_W // _CHUNK


def _vq_tc_body(x_ref, cb_ref, idx_ref, loss_ref, accp_ref, accs_ref):
    i = pl.program_id(0)
    x = x_ref[...]
    cb = cb_ref[...]
    mm = lax.dot_general(x, cb, (((1,), (1,)), ((), ())),
                         preferred_element_type=jnp.float32)
    x2 = jnp.sum(x * x, axis=1, keepdims=True)
    c2 = jnp.sum(cb * cb, axis=1)[None, :]
    d2 = jnp.maximum(x2 + c2 - 2.0 * mm, 0.0)
    d = jnp.sqrt(d2)

    dmin = jnp.min(d, axis=1, keepdims=True)
    kio = lax.broadcasted_iota(jnp.int32, (_TILE, _K), 1)
    idx = jnp.min(jnp.where(d == dmin, kio, _K), axis=1)
    idx_ref[...] = idx.reshape(1, 1, _TILE)

    sh = dmin - d
    p = jnp.exp(sh)
    s = jnp.sum(p, axis=1, keepdims=True)
    t = jnp.sum(p * sh, axis=1, keepdims=True)
    samp = jnp.sum(t / s - jnp.log(s))
    sq = jnp.sum(dmin * dmin)
    pcol = jnp.sum(p * (1.0 / s), axis=0, keepdims=True)

    @pl.when(i == 0)
    def _():
        accp_ref[...] = jnp.zeros_like(accp_ref)
        accs_ref[0] = 0.0
        accs_ref[1] = 0.0

    accp_ref[...] += pcol
    accs_ref[0] += sq
    accs_ref[1] += samp

    @pl.when(i == _NTILES - 1)
    def _():
        avgp = accp_ref[...] / _N
        avg_ent = -jnp.sum(avgp * jnp.log(avgp + 1e-5))
        sample_ent = -(accs_ref[1] / _N)
        latent = (1.0 + _COMMITMENT_COST) * (accs_ref[0] / (_N * _D))
        loss = latent + _ENTROPY_LOSS_RATIO * (sample_ent - avg_ent)
        loss_ref[...] = jnp.full((1, 128), loss, jnp.float32)


_vq_tc = pl.pallas_call(
    _vq_tc_body,
    grid=(_NTILES,),
    in_specs=[
        pl.BlockSpec((_TILE, _D), lambda i: (i, 0)),
        pl.BlockSpec((_K, _D), lambda i: (0, 0)),
    ],
    out_specs=[
        pl.BlockSpec((1, 1, _TILE), lambda i: (i, 0, 0)),
        pl.BlockSpec((1, 128), lambda i: (0, 0)),
    ],
    out_shape=[
        jax.ShapeDtypeStruct((_NTILES, 1, _TILE), jnp.int32),
        jax.ShapeDtypeStruct((1, 128), jnp.float32),
    ],
    scratch_shapes=[
        pltpu.VMEM((1, _K), jnp.float32),
        pltpu.SMEM((2,), jnp.float32),
    ],
)


_DPAD = 128


def _sc_gather_body(cb_hbm, idx_hbm, out_hbm, idx_v, rows_v, sem):
    c = lax.axis_index("c")
    s = lax.axis_index("s")
    wid = s * 2 + c
    base = wid * _ROWS_PER_W
    pltpu.sync_copy(idx_hbm.at[wid], idx_v)
    for j in range(_NCHUNK):
        pltpu.async_copy(cb_hbm.at[idx_v.at[j]], rows_v, sem).wait()
        pltpu.sync_copy(rows_v, out_hbm.at[pl.ds(base + j * _CHUNK, _CHUNK)])


_sc_gather_cache = []


def _sc_gather(codebook, idx3):
    if not _sc_gather_cache:
        _sc_gather_cache.append(pl.kernel(
            _sc_gather_body,
            out_type=jax.ShapeDtypeStruct((_N, _DPAD), jnp.float32),
            mesh=plsc.VectorSubcoreMesh(core_axis_name="c",
                                        subcore_axis_name="s"),
            scratch_types=[
                pltpu.VMEM((_NCHUNK, _CHUNK), jnp.int32),
                pltpu.VMEM((_CHUNK, _DPAD), jnp.float32),
                pltpu.SemaphoreType.DMA,
            ],
        ))
    return _sc_gather_cache[0](codebook, idx3)


def kernel(x, codebook):
    flat_x = x.reshape(_N, _D)
    idx3, loss_out = _vq_tc(flat_x, codebook)
    idx = idx3.reshape(_N)
    cb_pad = jnp.concatenate(
        [codebook, jnp.zeros((_K, _DPAD - _D), jnp.float32)], axis=1)
    quantized = _sc_gather(cb_pad, idx.reshape(_NW, _NCHUNK, _CHUNK))
    return quantized[:, :_D].reshape(x.shape), loss_out[0, 0], idx


# speedup vs baseline: 2.6463x; 1.3787x over previous
import jax
import jax.numpy as jnp
from jax import lax
from jax.experimental import pallas as pl
from jax.experimental.pallas import tpu as pltpu
from jax.experimental.pallas import tpu_sc as plsc

_B, _T, _D = 16, 576, 64
_K = 1024
_N = _B * _T
_TILE = 1152
_NTILES = _N // _TILE
_COMMITMENT_COST = 0.25
_ENTROPY_LOSS_RATIO = 0.1

_NW = 32
_ROWS_PER_W = _N // _NW
_CHUNK = 96
_NCHUNK = _ROWS_PER_W // _CHUNK


def _vq_tc_body(x_ref, cb_ref, idx_ref, loss_ref, q_ref, accp_ref, accs_ref):
    i = pl.program_id(0)
    x = x_ref[...]
    cb = cb_ref[...]
    mm = lax.dot_general(x, cb, (((1,), (1,)), ((), ())),
                         preferred_element_type=jnp.float32)
    x2 = jnp.sum(x * x, axis=1, keepdims=True)
    c2 = jnp.sum(cb * cb, axis=1)[None, :]
    d2 = jnp.maximum(x2 + c2 - 2.0 * mm, 0.0)
    d = jnp.sqrt(d2)

    dmin = jnp.min(d, axis=1, keepdims=True)
    kio = lax.broadcasted_iota(jnp.int32, (_TILE, _K), 1)
    idx = jnp.min(jnp.where(d == dmin, kio, _K), axis=1)
    idx_ref[...] = idx.reshape(1, 1, _TILE)
    onehot = (kio == idx[:, None]).astype(jnp.float32)
    q_ref[...] = lax.dot_general(onehot, cb, (((1,), (0,)), ((), ())),
                                 preferred_element_type=jnp.float32)

    sh = dmin - d
    p = jnp.exp(sh)
    s = jnp.sum(p, axis=1, keepdims=True)
    t = jnp.sum(p * sh, axis=1, keepdims=True)
    samp = jnp.sum(t / s - jnp.log(s))
    sq = jnp.sum(dmin * dmin)
    pcol = jnp.sum(p * (1.0 / s), axis=0, keepdims=True)

    @pl.when(i == 0)
    def _():
        accp_ref[...] = jnp.zeros_like(accp_ref)
        accs_ref[0] = 0.0
        accs_ref[1] = 0.0

    accp_ref[...] += pcol
    accs_ref[0] += sq
    accs_ref[1] += samp

    @pl.when(i == _NTILES - 1)
    def _():
        avgp = accp_ref[...] / _N
        avg_ent = -jnp.sum(avgp * jnp.log(avgp + 1e-5))
        sample_ent = -(accs_ref[1] / _N)
        latent = (1.0 + _COMMITMENT_COST) * (accs_ref[0] / (_N * _D))
        loss = latent + _ENTROPY_LOSS_RATIO * (sample_ent - avg_ent)
        loss_ref[...] = jnp.full((1, 128), loss, jnp.float32)


_vq_tc = pl.pallas_call(
    _vq_tc_body,
    grid=(_NTILES,),
    in_specs=[
        pl.BlockSpec((_TILE, _D), lambda i: (i, 0)),
        pl.BlockSpec((_K, _D), lambda i: (0, 0)),
    ],
    out_specs=[
        pl.BlockSpec((1, 1, _TILE), lambda i: (i, 0, 0)),
        pl.BlockSpec((1, 128), lambda i: (0, 0)),
        pl.BlockSpec((_TILE, _D), lambda i: (i, 0)),
    ],
    out_shape=[
        jax.ShapeDtypeStruct((_NTILES, 1, _TILE), jnp.int32),
        jax.ShapeDtypeStruct((1, 128), jnp.float32),
        jax.ShapeDtypeStruct((_N, _D), jnp.float32),
    ],
    scratch_shapes=[
        pltpu.VMEM((1, _K), jnp.float32),
        pltpu.SMEM((2,), jnp.float32),
    ],
)


_DPAD = 128


def _sc_gather_body(cb_hbm, idx_hbm, out_hbm, idx_v, rows_v, sem):
    c = lax.axis_index("c")
    s = lax.axis_index("s")
    wid = s * 2 + c
    base = wid * _ROWS_PER_W
    pltpu.sync_copy(idx_hbm.at[wid], idx_v)
    for j in range(_NCHUNK):
        pltpu.async_copy(cb_hbm.at[idx_v.at[j]], rows_v, sem).wait()
        pltpu.sync_copy(rows_v, out_hbm.at[pl.ds(base + j * _CHUNK, _CHUNK)])


_sc_gather_cache = []


def _sc_gather(codebook, idx3):
    if not _sc_gather_cache:
        _sc_gather_cache.append(pl.kernel(
            _sc_gather_body,
            out_type=jax.ShapeDtypeStruct((_N, _DPAD), jnp.float32),
            mesh=plsc.VectorSubcoreMesh(core_axis_name="c",
                                        subcore_axis_name="s"),
            scratch_types=[
                pltpu.VMEM((_NCHUNK, _CHUNK), jnp.int32),
                pltpu.VMEM((_CHUNK, _DPAD), jnp.float32),
                pltpu.SemaphoreType.DMA,
            ],
        ))
    return _sc_gather_cache[0](codebook, idx3)


def kernel(x, codebook):
    flat_x = x.reshape(_N, _D)
    idx3, loss_out, quantized = _vq_tc(flat_x, codebook)
    idx = idx3.reshape(_N)
    return quantized.reshape(x.shape), loss_out[0, 0], idx
